# baseline (speedup 1.0000x reference)
import jax
import jax.numpy as jnp
from jax.experimental import pallas as pl
from jax.experimental.pallas import tpu as pltpu

_BM = 200
_NBUF = 5


def _gcn_body(adj_hbm, x_hbm, w1_ref, w2_ref, w3_ref,
              mu_ref, lv_ref, bufs, x_ref, s1_ref, s23_ref, w23_ref,
              sems, xsem):
    p = pl.program_id(0)
    i = pl.program_id(1)
    nb = pl.num_programs(1)
    g = p * nb + i
    total = 2 * nb

    def issue(step):
        blk = jax.lax.rem(step, nb)
        slot = jax.lax.rem(step, _NBUF)
        pltpu.make_async_copy(
            adj_hbm.at[pl.ds(blk * _BM, _BM), :],
            bufs.at[slot],
            sems.at[slot],
        ).start()

    @pl.when(g == 0)
    def _():
        for s in range(_NBUF - 1):
            issue(jnp.int32(s))
        cp = pltpu.make_async_copy(x_hbm, x_ref, xsem)
        cp.start()
        cp.wait()
        s1_ref[...] = jnp.dot(x_ref[...], w1_ref[...],
                              preferred_element_type=jnp.float32)
        h = w2_ref.shape[1]
        w23_ref[:, :h] = w2_ref[...]
        w23_ref[:, h:] = w3_ref[...]

    @pl.when(g + _NBUF - 1 < total)
    def _():
        issue(g + _NBUF - 1)

    slot = jax.lax.rem(g, _NBUF)
    pltpu.make_async_copy(
        adj_hbm.at[pl.ds(0, _BM), :], bufs.at[slot], sems.at[slot]
    ).wait()

    @pl.when(p == 0)
    def _():
        h1_blk = jnp.maximum(
            jnp.dot(bufs[slot], s1_ref[...],
                    preferred_element_type=jnp.float32), 0.0)
        s23_ref[pl.ds(i * _BM, _BM), :] = jnp.dot(
            h1_blk, w23_ref[...], preferred_element_type=jnp.float32)

    @pl.when(p == 1)
    def _():
        blk = jnp.maximum(
            jnp.dot(bufs[slot], s23_ref[...],
                    preferred_element_type=jnp.float32), 0.0)
        h = mu_ref.shape[1]
        mu_ref[...] = blk[:, :h]
        lv_ref[...] = blk[:, h:]


def kernel(x, adj, W1, W2, W3):
    n, d = x.shape
    h1w = W1.shape[1]
    h2 = W2.shape[1]
    nb = n // _BM

    out_idx = lambda p, i: (i * p, 0)

    mu, logvar = pl.pallas_call(
        _gcn_body,
        grid=(2, nb),
        in_specs=[
            pl.BlockSpec(memory_space=pl.ANY),
            pl.BlockSpec(memory_space=pl.ANY),
            pl.BlockSpec((d, h1w), lambda p, i: (0, 0)),
            pl.BlockSpec((h1w, h2), lambda p, i: (0, 0)),
            pl.BlockSpec((h1w, h2), lambda p, i: (0, 0)),
        ],
        out_specs=[
            pl.BlockSpec((_BM, h2), out_idx),
            pl.BlockSpec((_BM, h2), out_idx),
        ],
        out_shape=[
            jax.ShapeDtypeStruct((n, h2), jnp.float32),
            jax.ShapeDtypeStruct((n, h2), jnp.float32),
        ],
        scratch_shapes=[
            pltpu.VMEM((_NBUF, _BM, n), jnp.float32),
            pltpu.VMEM((n, d), jnp.float32),
            pltpu.VMEM((n, h1w), jnp.float32),
            pltpu.VMEM((n, 2 * h2), jnp.float32),
            pltpu.VMEM((h1w, 2 * h2), jnp.float32),
            pltpu.SemaphoreType.DMA((_NBUF,)),
            pltpu.SemaphoreType.DMA,
        ],
    )(adj, x, W1, W2, W3)
    return (mu, mu, logvar)
